# baseline (speedup 1.0000x reference)
import functools

import jax
import jax.numpy as jnp
from jax import lax
from jax.experimental import pallas as pl
from jax.experimental.pallas import tpu as pltpu
from jax.experimental.pallas import tpu_sc as plsc

E = 64
K = 2
D_MODEL = 768
D_FF = 2048
T = 2048

BM = 128
S = T * K // BM + E
RP = S * BM
BF = 2048
NF = D_FF // BF

NC = 2
NS = 16
NW = NC * NS
TPW = T // NW
DH = D_MODEL // 2


def _router_kernel(x_ref, wg_ref, di_ref, wt_ref, meta_ref, xb_ref):
    x = x_ref[...]
    logits = jnp.dot(x, wg_ref[...], preferred_element_type=jnp.float32)
    lane = lax.broadcasted_iota(jnp.int32, (T, E), 1).astype(jnp.float32)

    m1 = jnp.max(logits, axis=1, keepdims=True)
    e1 = jnp.min(jnp.where(logits == m1, lane, float(E)), axis=1, keepdims=True)
    masked = jnp.where(lane == e1, -jnp.inf, logits)
    m2 = jnp.max(masked, axis=1, keepdims=True)
    e2 = jnp.min(jnp.where(masked == m2, lane, float(E)), axis=1, keepdims=True)

    t = jnp.exp(m2 - m1)
    w_hi = 1.0 / (1.0 + t)
    w_lo = t / (1.0 + t)

    H1 = (lane == e1).astype(jnp.float32)
    H2 = (lane == e2).astype(jnp.float32)
    Hs = H1 + H2

    counts_row = jnp.sum(Hs, axis=0, keepdims=True)
    ones_col = jnp.full((T, 1), 1.0, dtype=jnp.float32)
    counts_col = lax.dot_general(Hs, ones_col, (((0,), (0,)), ((), ())),
                                 preferred_element_type=jnp.float32)

    ntiles_row = jnp.floor((counts_row + (BM - 1)) * (1.0 / BM))
    ntiles_col = jnp.floor((counts_col + (BM - 1)) * (1.0 / BM))
    ei = lax.broadcasted_iota(jnp.int32, (E, E), 0).astype(jnp.float32)
    ej = lax.broadcasted_iota(jnp.int32, (E, E), 1).astype(jnp.float32)
    tri_u = (ei < ej).astype(jnp.float32)
    tri_l = (ei > ej).astype(jnp.float32)
    cum_row = jnp.dot(ntiles_row, tri_u, preferred_element_type=jnp.float32)
    cum_col = jnp.dot(tri_l, ntiles_col, preferred_element_type=jnp.float32)
    starts_row = BM * cum_row
    total_tiles = jnp.sum(ntiles_row, axis=1, keepdims=True)

    s_iota = lax.broadcasted_iota(jnp.int32, (1, 128), 1).astype(jnp.float32)
    s_real = jnp.minimum(s_iota, total_tiles - 1.0)
    ge = (s_real >= cum_col).astype(jnp.float32)
    gid_row = jnp.sum(ge, axis=0, keepdims=True) - 1.0

    CH = 512
    ci = lax.broadcasted_iota(jnp.int32, (CH, CH), 0).astype(jnp.float32)
    cj = lax.broadcasted_iota(jnp.int32, (CH, CH), 1).astype(jnp.float32)
    Lc = (ci > cj).astype(jnp.float32)
    carry = jnp.zeros((1, E), dtype=jnp.float32)
    parts = []
    for c in range(T // CH):
        Hc = Hs[c * CH:(c + 1) * CH, :]
        parts.append(jnp.dot(Lc, Hc, preferred_element_type=jnp.float32) + carry)
        carry = carry + jnp.sum(Hc, axis=0, keepdims=True)
    P = jnp.concatenate(parts, axis=0)

    rank_hi = jnp.sum(P * H1, axis=1, keepdims=True)
    rank_lo = jnp.sum(P * H2, axis=1, keepdims=True)
    start_hi = jnp.sum(H1 * starts_row, axis=1, keepdims=True)
    start_lo = jnp.sum(H2 * starts_row, axis=1, keepdims=True)
    d_hi = start_hi + rank_hi
    d_lo = start_lo + rank_lo

    di_ref[...] = jnp.zeros((T, 8), dtype=jnp.int32)
    di_ref[:, 0:1] = d_hi.astype(jnp.int32)
    di_ref[:, 1:2] = d_lo.astype(jnp.int32)
    wt_ref[...] = jnp.zeros((T, 8), dtype=jnp.float32)
    wt_ref[:, 0:1] = w_hi
    wt_ref[:, 1:2] = w_lo
    meta_ref[...] = jnp.zeros((8, 128), dtype=jnp.int32)
    meta_ref[0:1, 0:E] = starts_row.astype(jnp.int32)
    meta_ref[1:2, 0:E] = counts_row.astype(jnp.int32)
    meta_ref[2:3, :] = s_real.astype(jnp.int32)
    meta_ref[3:4, :] = gid_row.astype(jnp.int32)
    xb_ref[...] = x.astype(jnp.bfloat16)


def _gmm_kernel(gid_ref, sid_ref, starts_ref, counts_ref,
                x_ref, w1_ref, w3_ref, w2_ref, out_ref):
    s = pl.program_id(0)

    @pl.when(sid_ref[s] == s)
    def _():
        g = gid_ref[s]
        xb = x_ref[...]
        h = jnp.dot(xb, w1_ref[0].astype(jnp.bfloat16),
                    preferred_element_type=jnp.float32)
        gg = jnp.dot(xb, w3_ref[0].astype(jnp.bfloat16),
                     preferred_element_type=jnp.float32)
        a = (h / (1.0 + jnp.exp(-h))) * gg
        o = jnp.dot(a.astype(jnp.bfloat16), w2_ref[0].astype(jnp.bfloat16),
                    preferred_element_type=jnp.float32)
        row = lax.broadcasted_iota(jnp.int32, (BM, 1), 0)
        nvalid = counts_ref[g] - (sid_ref[s] * BM - starts_ref[g])
        out_ref[...] = jnp.where(row < nvalid, o, 0.0).astype(jnp.bfloat16)


def _combine_kernel(a_ref, b_ref, wt_ref, o_ref):
    a = a_ref[...].astype(jnp.float32)
    b = b_ref[...].astype(jnp.float32)
    o_ref[...] = a * wt_ref[:, 0:1] + b * wt_ref[:, 1:2]


def _scatter_x_body(x_hbm, de_hbm, do_hbm, out_hbm, idx_e, idx_o, rows, sem_e, sem_o):
    wid = lax.axis_index("s") * NC + lax.axis_index("c")
    base = wid * TPW
    pltpu.sync_copy(x_hbm.at[pl.ds(base, TPW), :], rows)
    pltpu.sync_copy(de_hbm.at[pl.ds(base, TPW)], idx_e)
    pltpu.sync_copy(do_hbm.at[pl.ds(base, TPW)], idx_o)
    ce = pltpu.async_copy(rows, out_hbm.at[idx_e], sem_e)
    co = pltpu.async_copy(rows, out_hbm.at[idx_o], sem_o)
    ce.wait()
    co.wait()


def _gather_out_body(osort_hbm, de_hbm, do_hbm, a_hbm, b_hbm,
                     idx_e, idx_o, rows_e, rows_o, sem_e, sem_o):
    wid = lax.axis_index("s") * NC + lax.axis_index("c")
    base = wid * TPW
    pltpu.sync_copy(de_hbm.at[pl.ds(base, TPW)], idx_e)
    pltpu.sync_copy(do_hbm.at[pl.ds(base, TPW)], idx_o)
    ce = pltpu.async_copy(osort_hbm.at[idx_e], rows_e, sem_e)
    co = pltpu.async_copy(osort_hbm.at[idx_o], rows_o, sem_o)
    ce.wait()
    co.wait()
    pltpu.sync_copy(rows_e, a_hbm.at[pl.ds(base, TPW), :])
    pltpu.sync_copy(rows_o, b_hbm.at[pl.ds(base, TPW), :])


@functools.cache
def _sc_kernels():
    mesh = plsc.VectorSubcoreMesh(core_axis_name="c", subcore_axis_name="s",
                                  num_cores=NC, num_subcores=NS)
    scatter_x = pl.kernel(
        _scatter_x_body,
        out_type=jax.ShapeDtypeStruct((RP, DH), jnp.float32),
        mesh=mesh,
        scratch_types=[
            pltpu.VMEM((TPW,), jnp.int32),
            pltpu.VMEM((TPW,), jnp.int32),
            pltpu.VMEM((TPW, DH), jnp.float32),
            pltpu.SemaphoreType.DMA,
            pltpu.SemaphoreType.DMA,
        ],
    )
    gather_out = pl.kernel(
        _gather_out_body,
        out_type=(jax.ShapeDtypeStruct((T, DH), jnp.float32),
                  jax.ShapeDtypeStruct((T, DH), jnp.float32)),
        mesh=mesh,
        scratch_types=[
            pltpu.VMEM((TPW,), jnp.int32),
            pltpu.VMEM((TPW,), jnp.int32),
            pltpu.VMEM((TPW, DH), jnp.float32),
            pltpu.VMEM((TPW, DH), jnp.float32),
            pltpu.SemaphoreType.DMA,
            pltpu.SemaphoreType.DMA,
        ],
    )
    return scatter_x, gather_out


def _router_call(x, W_gate):
    return pl.pallas_call(
        _router_kernel,
        out_shape=[
            jax.ShapeDtypeStruct((T, 8), jnp.int32),
            jax.ShapeDtypeStruct((T, 8), jnp.float32),
            jax.ShapeDtypeStruct((8, 128), jnp.int32),
            jax.ShapeDtypeStruct((T, D_MODEL), jnp.bfloat16),
        ],
    )(x, W_gate)


def _gmm_call(gid, sid, starts, counts, x_sorted, w1, w3, w2):
    grid_spec = pltpu.PrefetchScalarGridSpec(
        num_scalar_prefetch=4,
        grid=(S,),
        in_specs=[
            pl.BlockSpec((BM, D_MODEL), lambda s, gid, sid, st, ct: (sid[s], 0)),
            pl.BlockSpec((1, D_MODEL, BF), lambda s, gid, sid, st, ct: (gid[s], 0, 0)),
            pl.BlockSpec((1, D_MODEL, BF), lambda s, gid, sid, st, ct: (gid[s], 0, 0)),
            pl.BlockSpec((1, BF, D_MODEL), lambda s, gid, sid, st, ct: (gid[s], 0, 0)),
        ],
        out_specs=pl.BlockSpec((BM, D_MODEL), lambda s, gid, sid, st, ct: (sid[s], 0)),
    )
    return pl.pallas_call(
        _gmm_kernel,
        out_shape=jax.ShapeDtypeStruct((RP, D_MODEL), jnp.bfloat16),
        grid_spec=grid_spec,
        compiler_params=pltpu.CompilerParams(
            dimension_semantics=("arbitrary",),
            vmem_limit_bytes=100 * 1024 * 1024,
        ),
    )(gid, sid, starts, counts, x_sorted, w1, w3, w2)


def _combine_call(a, b, wt):
    BT = 512
    return pl.pallas_call(
        _combine_kernel,
        out_shape=jax.ShapeDtypeStruct((T, D_MODEL), jnp.float32),
        grid=(T // BT,),
        in_specs=[
            pl.BlockSpec((BT, D_MODEL), lambda i: (i, 0)),
            pl.BlockSpec((BT, D_MODEL), lambda i: (i, 0)),
            pl.BlockSpec((BT, 8), lambda i: (i, 0)),
        ],
        out_specs=pl.BlockSpec((BT, D_MODEL), lambda i: (i, 0)),
    )(a, b, wt)


def kernel(x, W_gate, w1, w2, w3):
    di, wt, meta, xb = _router_call(x, W_gate)
    d_hi = di[:, 0]
    d_lo = di[:, 1]
    starts = meta[0, :E]
    counts = meta[1, :E]
    sid = meta[2, :S]
    gid = meta[3, :S]
    scatter_x, gather_out = _sc_kernels()
    xf = lax.bitcast_convert_type(xb.reshape(T, DH, 2), jnp.float32)
    xs_f = scatter_x(xf, d_hi, d_lo)
    x_sorted = lax.bitcast_convert_type(xs_f, jnp.bfloat16).reshape(RP, D_MODEL)
    out_sort = _gmm_call(gid, sid, starts, counts, x_sorted, w1, w3, w2)
    os_f = lax.bitcast_convert_type(out_sort.reshape(RP, DH, 2), jnp.float32)
    a_f, b_f = gather_out(os_f, d_hi, d_lo)
    a = lax.bitcast_convert_type(a_f, jnp.bfloat16).reshape(T, D_MODEL)
    b = lax.bitcast_convert_type(b_f, jnp.bfloat16).reshape(T, D_MODEL)
    return _combine_call(a, b, wt)


# speedup vs baseline: 2.1915x; 2.1915x over previous
import functools

import jax
import jax.numpy as jnp
from jax import lax
from jax.experimental import pallas as pl
from jax.experimental.pallas import tpu as pltpu
from jax.experimental.pallas import tpu_sc as plsc

E = 64
K = 2
D_MODEL = 768
D_FF = 2048
T = 2048

BM = 128
S = T * K // BM + E
RP = S * BM
BF = 2048
NF = D_FF // BF

NC = 2
NS = 16
NW = NC * NS
TPW = T // NW
DH = D_MODEL // 2


def _router_kernel(x_ref, wg_ref, di_ref, wt_ref, meta_ref):
    x = x_ref[...]
    logits = jnp.dot(x, wg_ref[...], preferred_element_type=jnp.float32)
    lane = lax.broadcasted_iota(jnp.int32, (T, E), 1).astype(jnp.float32)

    m1 = jnp.max(logits, axis=1, keepdims=True)
    e1 = jnp.min(jnp.where(logits == m1, lane, float(E)), axis=1, keepdims=True)
    masked = jnp.where(lane == e1, -jnp.inf, logits)
    m2 = jnp.max(masked, axis=1, keepdims=True)
    e2 = jnp.min(jnp.where(masked == m2, lane, float(E)), axis=1, keepdims=True)

    t = jnp.exp(m2 - m1)
    w_hi = 1.0 / (1.0 + t)
    w_lo = t / (1.0 + t)

    H1 = (lane == e1).astype(jnp.float32)
    H2 = (lane == e2).astype(jnp.float32)
    Hs = H1 + H2

    counts_row = jnp.sum(Hs, axis=0, keepdims=True)
    ones_col = jnp.full((T, 1), 1.0, dtype=jnp.float32)
    counts_col = lax.dot_general(Hs, ones_col, (((0,), (0,)), ((), ())),
                                 preferred_element_type=jnp.float32)

    ntiles_row = jnp.floor((counts_row + (BM - 1)) * (1.0 / BM))
    ntiles_col = jnp.floor((counts_col + (BM - 1)) * (1.0 / BM))
    ei = lax.broadcasted_iota(jnp.int32, (E, E), 0).astype(jnp.float32)
    ej = lax.broadcasted_iota(jnp.int32, (E, E), 1).astype(jnp.float32)
    tri_u = (ei < ej).astype(jnp.float32)
    tri_l = (ei > ej).astype(jnp.float32)
    cum_row = jnp.dot(ntiles_row, tri_u, preferred_element_type=jnp.float32)
    cum_col = jnp.dot(tri_l, ntiles_col, preferred_element_type=jnp.float32)
    starts_row = BM * cum_row
    total_tiles = jnp.sum(ntiles_row, axis=1, keepdims=True)

    s_iota = lax.broadcasted_iota(jnp.int32, (1, 128), 1).astype(jnp.float32)
    s_real = jnp.minimum(s_iota, total_tiles - 1.0)
    ge = (s_real >= cum_col).astype(jnp.float32)
    gid_row = jnp.sum(ge, axis=0, keepdims=True) - 1.0

    CH = 512
    ci = lax.broadcasted_iota(jnp.int32, (CH, CH), 0).astype(jnp.float32)
    cj = lax.broadcasted_iota(jnp.int32, (CH, CH), 1).astype(jnp.float32)
    Lc = (ci > cj).astype(jnp.float32)
    carry = jnp.zeros((1, E), dtype=jnp.float32)
    parts = []
    for c in range(T // CH):
        Hc = Hs[c * CH:(c + 1) * CH, :]
        parts.append(jnp.dot(Lc, Hc, preferred_element_type=jnp.float32) + carry)
        carry = carry + jnp.sum(Hc, axis=0, keepdims=True)
    P = jnp.concatenate(parts, axis=0)

    rank_hi = jnp.sum(P * H1, axis=1, keepdims=True)
    rank_lo = jnp.sum(P * H2, axis=1, keepdims=True)
    start_hi = jnp.sum(H1 * starts_row, axis=1, keepdims=True)
    start_lo = jnp.sum(H2 * starts_row, axis=1, keepdims=True)
    d_hi = start_hi + rank_hi
    d_lo = start_lo + rank_lo

    di_ref[...] = jnp.zeros((T, 8), dtype=jnp.int32)
    di_ref[:, 0:1] = d_hi.astype(jnp.int32)
    di_ref[:, 1:2] = d_lo.astype(jnp.int32)
    wt_ref[...] = jnp.zeros((T, 8), dtype=jnp.float32)
    wt_ref[:, 0:1] = w_hi
    wt_ref[:, 1:2] = w_lo
    meta_ref[...] = jnp.zeros((8, 128), dtype=jnp.int32)
    meta_ref[0:1, 0:E] = starts_row.astype(jnp.int32)
    meta_ref[1:2, 0:E] = counts_row.astype(jnp.int32)
    meta_ref[2:3, :] = s_real.astype(jnp.int32)
    meta_ref[3:4, :] = gid_row.astype(jnp.int32)


def _gmm_kernel(gid_ref, sid_ref, starts_ref, counts_ref,
                x_ref, w1_ref, w3_ref, w2_ref, out_ref):
    s = pl.program_id(0)

    @pl.when(sid_ref[s] == s)
    def _():
        g = gid_ref[s]
        xb = x_ref[...].astype(jnp.bfloat16)
        h = jnp.dot(xb, w1_ref[0].astype(jnp.bfloat16),
                    preferred_element_type=jnp.float32)
        gg = jnp.dot(xb, w3_ref[0].astype(jnp.bfloat16),
                     preferred_element_type=jnp.float32)
        a = (h / (1.0 + jnp.exp(-h))) * gg
        o = jnp.dot(a.astype(jnp.bfloat16), w2_ref[0].astype(jnp.bfloat16),
                    preferred_element_type=jnp.float32)
        row = lax.broadcasted_iota(jnp.int32, (BM, 1), 0)
        nvalid = counts_ref[g] - (sid_ref[s] * BM - starts_ref[g])
        out_ref[...] = jnp.where(row < nvalid, o, 0.0)


def _combine_kernel(a_ref, b_ref, wt_ref, o_ref):
    o_ref[...] = (a_ref[...] * wt_ref[:, 0:1] + b_ref[...] * wt_ref[:, 1:2])


def _scatter_x_body(x_hbm, de_hbm, do_hbm, out_hbm, idx_e, idx_o, rows, sem_e, sem_o):
    wid = lax.axis_index("s") * NC + lax.axis_index("c")
    base = wid * TPW
    pltpu.sync_copy(x_hbm.at[pl.ds(base, TPW), :], rows)
    pltpu.sync_copy(de_hbm.at[pl.ds(base, TPW)], idx_e)
    pltpu.sync_copy(do_hbm.at[pl.ds(base, TPW)], idx_o)
    ce = pltpu.async_copy(rows, out_hbm.at[idx_e], sem_e)
    co = pltpu.async_copy(rows, out_hbm.at[idx_o], sem_o)
    ce.wait()
    co.wait()


def _gather_out_body(osort_hbm, de_hbm, do_hbm, a_hbm, b_hbm,
                     idx_e, idx_o, rows_e, rows_o, sem_e, sem_o):
    wid = lax.axis_index("s") * NC + lax.axis_index("c")
    base = wid * TPW
    pltpu.sync_copy(de_hbm.at[pl.ds(base, TPW)], idx_e)
    pltpu.sync_copy(do_hbm.at[pl.ds(base, TPW)], idx_o)
    ce = pltpu.async_copy(osort_hbm.at[idx_e], rows_e, sem_e)
    co = pltpu.async_copy(osort_hbm.at[idx_o], rows_o, sem_o)
    ce.wait()
    co.wait()
    pltpu.sync_copy(rows_e, a_hbm.at[pl.ds(base, TPW), :])
    pltpu.sync_copy(rows_o, b_hbm.at[pl.ds(base, TPW), :])


@functools.cache
def _sc_kernels():
    mesh = plsc.VectorSubcoreMesh(core_axis_name="c", subcore_axis_name="s",
                                  num_cores=NC, num_subcores=NS)
    scatter_x = pl.kernel(
        _scatter_x_body,
        out_type=jax.ShapeDtypeStruct((RP, D_MODEL), jnp.float32),
        mesh=mesh,
        scratch_types=[
            pltpu.VMEM((TPW,), jnp.int32),
            pltpu.VMEM((TPW,), jnp.int32),
            pltpu.VMEM((TPW, D_MODEL), jnp.float32),
            pltpu.SemaphoreType.DMA,
            pltpu.SemaphoreType.DMA,
        ],
    )
    gather_out = pl.kernel(
        _gather_out_body,
        out_type=(jax.ShapeDtypeStruct((T, D_MODEL), jnp.float32),
                  jax.ShapeDtypeStruct((T, D_MODEL), jnp.float32)),
        mesh=mesh,
        scratch_types=[
            pltpu.VMEM((TPW,), jnp.int32),
            pltpu.VMEM((TPW,), jnp.int32),
            pltpu.VMEM((TPW, D_MODEL), jnp.float32),
            pltpu.VMEM((TPW, D_MODEL), jnp.float32),
            pltpu.SemaphoreType.DMA,
            pltpu.SemaphoreType.DMA,
        ],
    )
    return scatter_x, gather_out


def _router_call(x, W_gate):
    return pl.pallas_call(
        _router_kernel,
        out_shape=[
            jax.ShapeDtypeStruct((T, 8), jnp.int32),
            jax.ShapeDtypeStruct((T, 8), jnp.float32),
            jax.ShapeDtypeStruct((8, 128), jnp.int32),
        ],
    )(x, W_gate)


def _gmm_call(gid, sid, starts, counts, x_sorted, w1, w3, w2):
    grid_spec = pltpu.PrefetchScalarGridSpec(
        num_scalar_prefetch=4,
        grid=(S,),
        in_specs=[
            pl.BlockSpec((BM, D_MODEL), lambda s, gid, sid, st, ct: (sid[s], 0)),
            pl.BlockSpec((1, D_MODEL, BF), lambda s, gid, sid, st, ct: (gid[s], 0, 0)),
            pl.BlockSpec((1, D_MODEL, BF), lambda s, gid, sid, st, ct: (gid[s], 0, 0)),
            pl.BlockSpec((1, BF, D_MODEL), lambda s, gid, sid, st, ct: (gid[s], 0, 0)),
        ],
        out_specs=pl.BlockSpec((BM, D_MODEL), lambda s, gid, sid, st, ct: (sid[s], 0)),
    )
    return pl.pallas_call(
        _gmm_kernel,
        out_shape=jax.ShapeDtypeStruct((RP, D_MODEL), jnp.float32),
        grid_spec=grid_spec,
        compiler_params=pltpu.CompilerParams(
            dimension_semantics=("arbitrary",),
            vmem_limit_bytes=100 * 1024 * 1024,
        ),
    )(gid, sid, starts, counts, x_sorted, w1, w3, w2)


def _combine_call(a, b, wt):
    BT = 512
    return pl.pallas_call(
        _combine_kernel,
        out_shape=jax.ShapeDtypeStruct((T, D_MODEL), jnp.float32),
        grid=(T // BT,),
        in_specs=[
            pl.BlockSpec((BT, D_MODEL), lambda i: (i, 0)),
            pl.BlockSpec((BT, D_MODEL), lambda i: (i, 0)),
            pl.BlockSpec((BT, 8), lambda i: (i, 0)),
        ],
        out_specs=pl.BlockSpec((BT, D_MODEL), lambda i: (i, 0)),
    )(a, b, wt)


def kernel(x, W_gate, w1, w2, w3):
    di, wt, meta = _router_call(x, W_gate)
    d_hi = di[:, 0]
    d_lo = di[:, 1]
    starts = meta[0, :E]
    counts = meta[1, :E]
    sid = meta[2, :S]
    gid = meta[3, :S]
    scatter_x, gather_out = _sc_kernels()
    x_sorted = scatter_x(x, d_hi, d_lo)
    out_sort = _gmm_call(gid, sid, starts, counts, x_sorted, w1, w3, w2)
    a, b = gather_out(out_sort, d_hi, d_lo)
    return _combine_call(a, b, wt)
